# Initial kernel scaffold; baseline (speedup 1.0000x reference)
import jax
import jax.numpy as jnp
from jax.experimental import pallas as pl


def kernel(x, edge_index, edge_attr, batch, params):
    raise NotImplementedError("write your pallas kernel here")


# speedup vs baseline: 1.0024x; 1.0024x over previous
import jax
import jax.numpy as jnp
from jax.experimental import pallas as pl

_N = 10000
_E = 160000
_NG = 64
_HID = 36
_CH = 18
_HEADS = 8
_NUM_LAYERS = 4


def _linear(p, x):
    y = x @ p["W"]
    if "b" in p:
        y = y + p["b"]
    return y


def _seg_softmax(alpha, dst, n):
    amax = jax.ops.segment_max(alpha, dst, num_segments=n)
    amax = jnp.where(jnp.isfinite(amax), amax, 0.0)
    a = jnp.exp(alpha - amax[dst])
    den = jax.ops.segment_sum(a, dst, num_segments=n)
    return a / (den[dst] + 1e-16)


def _gatv2(p, x, src, dst, edge_attr, heads, do, concat, n):
    xl = _linear(p["lin_l"], x).reshape(n, heads, do)
    xr = _linear(p["lin_r"], x).reshape(n, heads, do)
    ea = _linear(p["lin_edge"], edge_attr).reshape(-1, heads, do)
    e = xl[src] + xr[dst] + ea
    e = jax.nn.leaky_relu(e, 0.2)
    alpha = (e * p["att"][None]).sum(-1)
    alpha = _seg_softmax(alpha, dst, n)
    msg = xl[src] * alpha[..., None]
    out = jax.ops.segment_sum(msg, dst, num_segments=n)
    out = out.reshape(n, heads * do) if concat else out.mean(1)
    return out + p["bias"]


def _transformer(p, x, src, dst, edge_attr, heads, do, concat, n):
    q = _linear(p["lin_q"], x).reshape(n, heads, do)
    k = _linear(p["lin_k"], x).reshape(n, heads, do)
    v = _linear(p["lin_v"], x).reshape(n, heads, do)
    ea = _linear(p["lin_edge"], edge_attr).reshape(-1, heads, do)
    alpha = (q[dst] * (k[src] + ea)).sum(-1) / jnp.sqrt(float(do))
    alpha = _seg_softmax(alpha, dst, n)
    msg = (v[src] + ea) * alpha[..., None]
    out = jax.ops.segment_sum(msg, dst, num_segments=n)
    out = out.reshape(n, heads * do) if concat else out.mean(1)
    return out + _linear(p["lin_skip"], x)


def _graph_norm(p, x):
    mean = x.mean(0, keepdims=True)
    out = x - mean * p["ms"]
    var = (out * out).mean(0, keepdims=True)
    return p["w"] * out / jnp.sqrt(var + 1e-5) + p["b"]


def _layer_norm(p, x, eps):
    u = x.mean(-1, keepdims=True)
    s = ((x - u) ** 2).mean(-1, keepdims=True)
    return p["w"] * (x - u) / jnp.sqrt(s + eps) + p["b"]


def _identity_pallas(x):
    def body(x_ref, o_ref):
        o_ref[...] = x_ref[...]
    return pl.pallas_call(
        body, out_shape=jax.ShapeDtypeStruct(x.shape, x.dtype))(x)


def kernel(x, edge_index, edge_attr, batch, params):
    src = edge_index[0]
    dst = edge_index[1]
    n = x.shape[0]
    x = _gatv2(params["initial_conv"], x, src, dst, edge_attr, _HEADS, _HID, True, n)
    x = jax.nn.elu(_graph_norm(params["initial_norm"], x))
    for i in range(_NUM_LAYERS):
        lp = params["layers"][i]
        last = (i == _NUM_LAYERS - 1)
        do = _HID if not last else _CH
        heads = _HEADS if not last else 1
        concat = not last
        res = x
        xn = _graph_norm(lp["norm"], x)
        xg = jax.nn.elu(_gatv2(lp["conv"], xn, src, dst, edge_attr, heads, do, concat, n))
        xt = jax.nn.elu(_transformer(lp["transformer"], xn, src, dst, edge_attr, heads, do, concat, n))
        x = xg + xt
        if res.shape == x.shape:
            x = x + res
        res = x
        x = _layer_norm(lp["layer_norm"], x, 1e-6)
        x = jax.nn.elu(_linear(lp["lin"], x))
        if res.shape == x.shape:
            x = x + res
    pooled = jax.ops.segment_sum(x, batch, num_segments=_NG)
    out = _linear(params["final_lin"], pooled)
    out = _layer_norm(params["final_norm"], out, 1e-5)
    return jax.nn.elu(_identity_pallas(out))


# speedup vs baseline: 12.1068x; 12.0781x over previous
import functools

import jax
import jax.numpy as jnp
import numpy as np
from jax import lax
from jax.experimental import pallas as pl
from jax.experimental.pallas import tpu as pltpu
from jax.experimental.pallas import tpu_sc as plsc

N = 10000
E = 160000
NG = 64
CH = 18
NUM_LAYERS = 4

NCORES = 2
NSUB = 16
LN16 = 16

NPAD = 10240
RPT = NPAD // NSUB

def _mode_geom(mode):
    gat = mode.startswith("gat")
    wide = mode.endswith("8")
    chk = 48 if gat else 32
    if wide:
        per_tile = -(-E // NSUB)
        nw = NSUB
    else:
        per_tile = -(-E // (NSUB * NCORES))
        nw = NSUB * NCORES
    nch = -(-per_tile // chk)
    if nch % 2:
        nch += 1
    et = nch * chk
    return chk, et, nch, et * nw

F32 = jnp.float32
I32 = jnp.int32


def _sds(shape, dtype=F32):
    return jax.ShapeDtypeStruct(shape, dtype)


def _sc_mesh():
    return plsc.VectorSubcoreMesh(
        core_axis_name="c", subcore_axis_name="s",
        num_cores=NCORES, num_subcores=NSUB)


_IOTA_NP = np.arange(16, dtype=np.int32)
_GDN = lax.GatherDimensionNumbers(
    offset_dims=(), collapsed_slice_dims=(0,), start_index_map=(0,))


def _bfly_sum(v):
    io = lax.iota(I32, LN16)
    for k in (8, 4, 2, 1):
        perm = (io ^ k).reshape(LN16, 1)
        v = v + lax.gather(v, perm, _GDN, (1,),
                           mode=lax.GatherScatterMode.PROMISE_IN_BOUNDS)
    return v


def _build_sc_conv(mode):
    gat = mode.startswith("gat")
    wide = mode.endswith("8")
    D = 36 if wide else 18
    NHP = 2 if wide else 1
    MC = NHP * D
    NV = 5 if wide else 2
    npass = 2 if wide else 1
    CHK, ET, nch, _etot = _mode_geom(mode)
    npairs = nch // 2
    scale = 1.0 if gat else (1.0 / 6.0 if wide else float(1.0 / np.sqrt(18.0)))
    nparts = 4 if wide else 1
    outparts = 4 if wide else 2

    def body(*refs):
        if gat:
            (tabL, tabR, srcH, dstH, attrH, attPH, wePH, zrowsH, out_hbm) = refs[:9]
            rest = refs[9:]
            (attP_v, weP_v, is0, is1, idg0, idg1, idr0, idr1, at0, at1,
             bA0, bA1, bB0, bB1, msg, acc_sh, sem0, sem1) = rest
            bC0 = bC1 = None
        else:
            (tabQ, tabK, tabV, srcH, dstH, attrH, wePH, zrowsH, out_hbm) = refs[:9]
            rest = refs[9:]
            (weP_v, is0, is1, idg0, idg1, idr0, idr1, at0, at1,
             bA0, bA1, bB0, bB1, bC0, bC1, msg, acc_sh, sem0, sem1) = rest
            attP_v = None

        cid = lax.axis_index("c")
        sid = lax.axis_index("s")
        acc_base = sid * RPT
        io = lax.iota(I32, LN16)

        if gat:
            pltpu.sync_copy(attPH, attP_v)
        pltpu.sync_copy(wePH, weP_v)
        pltpu.sync_copy(zrowsH.at[pl.ds(0, CHK)], msg)

        if wide:
            ebase_tile = sid * ET
        else:
            ebase_tile = (sid * NCORES + cid) * ET

        def load_idx(k, is_, idg, idr, at, goff):
            eb = ebase_tile + k * CHK
            pltpu.sync_copy(srcH.at[pl.ds(eb, CHK)], is_)
            pltpu.sync_copy(dstH.at[pl.ds(eb, CHK)], idr)
            pltpu.sync_copy(attrH.at[pl.ds(eb, CHK)], at)

            def adj(g, _):
                sl = pl.ds(g * LN16, LN16)
                if wide:
                    is_[sl] = is_[sl] + goff
                idg[sl] = idr[sl] + goff
                return 0
            lax.fori_loop(0, CHK // LN16, adj, 0)

        def start_gathers(is_, idg, bA, bB, bC, sem):
            if gat:
                pltpu.async_copy(tabL.at[is_], bA, sem)
                pltpu.async_copy(tabR.at[idg], bB, sem)
            else:
                pltpu.async_copy(tabK.at[is_], bA, sem)
                pltpu.async_copy(tabQ.at[idg], bB, sem)
                pltpu.async_copy(tabV.at[is_], bC, sem)

        def wait_gathers(is_, idg, bA, bB, bC, sem):
            if gat:
                pltpu.make_async_copy(tabL.at[is_], bA, sem).wait()
                pltpu.make_async_copy(tabR.at[idg], bB, sem).wait()
            else:
                pltpu.make_async_copy(tabK.at[is_], bA, sem).wait()
                pltpu.make_async_copy(tabQ.at[idg], bB, sem).wait()
                pltpu.make_async_copy(tabV.at[is_], bC, sem).wait()

        if wide:
            m_h0v2 = io < 4
            den_lane0, den_lane1 = 8, 9
        else:
            m_h0v1 = io < 2
            den_lane0 = 2

        def compute_scatter(k, is_, idr, at, bA, bB, bC, consts):
            eb = ebase_tile + k * CHK
            attv = consts[:NV]
            wev = consts[NV:]

            def edge_body(e, consts_c):
                av = at[e, pl.ds(0, LN16)]
                a0 = av[0]
                a1 = av[1]
                a2 = av[2]
                ts = []
                mraw = []
                for j in range(NV):
                    sl = pl.ds(j * LN16, LN16)
                    ea = (a0 * wev[j] + a1 * wev[NV + j] + a2 * wev[2 * NV + j])
                    va = bA[e, sl]
                    vb = bB[e, sl]
                    if gat:
                        z = va + vb + ea
                        z = jnp.maximum(z, 0.0) + 0.2 * jnp.minimum(z, 0.0)
                        ts.append(attv[j] * z)
                        mraw.append(va)
                    else:
                        ts.append(vb * (va + ea))
                        mraw.append(bC[e, sl] + ea)
                if wide:
                    h0 = ts[0] + ts[1] + jnp.where(m_h0v2, ts[2], 0.0)
                    h1 = (ts[2] - jnp.where(m_h0v2, ts[2], 0.0)) + ts[3] + ts[4]
                else:
                    h0 = ts[0] + jnp.where(m_h0v1, ts[1], 0.0)
                    h1 = None
                eid = io * 0 + (eb + e)
                emask = jnp.where(eid < E, 1.0, 0.0)
                w0 = jnp.exp(_bfly_sum(h0) * scale) * emask
                if wide:
                    w1 = jnp.exp(_bfly_sum(h1) * scale) * emask
                if wide:
                    msg[e, pl.ds(0, LN16)] = mraw[0] * w0
                    msg[e, pl.ds(16, LN16)] = mraw[1] * w0
                    msg[e, pl.ds(32, LN16)] = jnp.where(
                        m_h0v2, mraw[2] * w0, mraw[2] * w1)
                    msg[e, pl.ds(48, LN16)] = mraw[3] * w1
                    dv = jnp.where(io == den_lane0, w0,
                                   jnp.where(io == den_lane1, w1, 0.0))
                    msg[e, pl.ds(64, LN16)] = jnp.where(
                        io < 8, mraw[4] * w1, dv)
                else:
                    msg[e, pl.ds(0, LN16)] = mraw[0] * w0
                    dv = jnp.where(io == den_lane0, w0, 0.0)
                    msg[e, pl.ds(16, LN16)] = jnp.where(
                        m_h0v1, mraw[1] * w0, dv)
                return consts_c

            lax.fori_loop(0, CHK, edge_body, consts)
            pltpu.sync_copy(msg, acc_sh.at[idr], add=True)

        for p in range(npass):
            q = cid + 2 * p if wide else 0
            goff = q * N

            pltpu.sync_copy(zrowsH, acc_sh.at[pl.ds(acc_base, RPT)])
            plsc.subcore_barrier()

            consts = []
            if gat:
                for j in range(NV):
                    consts.append(attP_v[q, pl.ds(j * LN16, LN16)])
            else:
                consts += [jnp.zeros((LN16,), F32)] * NV
            for jj in range(3):
                for j in range(NV):
                    consts.append(
                        weP_v[q, pl.ds(jj * NV * LN16 + j * LN16, LN16)])
            consts = tuple(consts)

            load_idx(0, is0, idg0, idr0, at0, goff)
            start_gathers(is0, idg0, bA0, bB0, bC0, sem0)

            def pair(g, consts_c):
                k0 = g * 2
                load_idx(k0 + 1, is1, idg1, idr1, at1, goff)
                start_gathers(is1, idg1, bA1, bB1, bC1, sem1)
                wait_gathers(is0, idg0, bA0, bB0, bC0, sem0)
                compute_scatter(k0, is0, idr0, at0, bA0, bB0, bC0, consts_c)

                @pl.when(g < npairs - 1)
                def _():
                    load_idx(k0 + 2, is0, idg0, idr0, at0, goff)
                    start_gathers(is0, idg0, bA0, bB0, bC0, sem0)

                wait_gathers(is1, idg1, bA1, bB1, bC1, sem1)
                compute_scatter(k0 + 1, is1, idr1, at1, bA1, bB1, bC1,
                                consts_c)
                return consts_c

            lax.fori_loop(0, npairs, pair, consts)
            plsc.subcore_barrier()

            opart = q if wide else cid
            pltpu.sync_copy(
                acc_sh.at[pl.ds(acc_base, RPT)],
                out_hbm.at[pl.ds(opart * NPAD + acc_base, RPT)])
            if p + 1 < npass:
                plsc.subcore_barrier()

    scratch = []
    if gat:
        scratch.append(pltpu.VMEM((nparts, NV * LN16), F32))
    scratch.append(pltpu.VMEM((nparts, 3 * NV * LN16), F32))
    scratch += [
        pltpu.VMEM((CHK,), I32), pltpu.VMEM((CHK,), I32),
        pltpu.VMEM((CHK,), I32), pltpu.VMEM((CHK,), I32),
        pltpu.VMEM((CHK,), I32), pltpu.VMEM((CHK,), I32),
        pltpu.VMEM((CHK, LN16), F32), pltpu.VMEM((CHK, LN16), F32),
        pltpu.VMEM((CHK, 128), F32), pltpu.VMEM((CHK, 128), F32),
        pltpu.VMEM((CHK, 128), F32), pltpu.VMEM((CHK, 128), F32),
    ]
    if not gat:
        scratch += [pltpu.VMEM((CHK, 128), F32), pltpu.VMEM((CHK, 128), F32)]
    scratch += [
        pltpu.VMEM((CHK, 128), F32),
        pltpu.VMEM_SHARED((NPAD, 128), F32),
        pltpu.SemaphoreType.DMA, pltpu.SemaphoreType.DMA,
    ]

    return pl.kernel(
        body,
        out_type=[_sds((outparts * NPAD, 128))],
        mesh=_sc_mesh(),
        scratch_types=scratch,
        name=f"sc_conv_{mode}",
    )


_SC_KERNELS = {}


def _sc_conv(mode, *args):
    if mode not in _SC_KERNELS:
        _SC_KERNELS[mode] = _build_sc_conv(mode)
    return _SC_KERNELS[mode](*args)[0]


_BR = 1000


def _elu(x):
    return jnp.where(x > 0, x, jnp.exp(jnp.minimum(x, 0.0)) - 1.0)


def _dot(a, b):
    return jnp.dot(a, b, preferred_element_type=F32,
                   precision=jax.lax.Precision.HIGHEST)


def _gn_from_stats(x, s1, s2, ms, w, b):
    mean = s1 * (1.0 / N)
    m2 = mean * ms
    var = s2 * (1.0 / N) - 2.0 * m2 * mean + m2 * m2
    return w * (x - m2) / jnp.sqrt(var + 1e-5) + b


def _assemble8(parts):
    cols = []
    for p in parts:
        cols.append(p[:, 0:36] / (p[:, 72:73] + 1e-16))
        cols.append(p[:, 36:72] / (p[:, 73:74] + 1e-16))
    return jnp.concatenate(cols, axis=1)


def _row_spec(c):
    return pl.BlockSpec((_BR, c), lambda i: (i, 0))


def _one_spec(c):
    return pl.BlockSpec((1, c), lambda i: (0, 0))


def _tc_initial_proj(x, p):
    def body(x_ref, wl, bl, wr, br, ol, orr):
        xv = x_ref[...]
        ol[...] = _dot(xv, wl[...]) + bl[...]
        orr[...] = _dot(xv, wr[...]) + br[...]
    return pl.pallas_call(
        body, grid=(N // _BR,),
        in_specs=[_row_spec(9),
                  pl.BlockSpec((9, 288), lambda i: (0, 0)), _one_spec(288),
                  pl.BlockSpec((9, 288), lambda i: (0, 0)), _one_spec(288)],
        out_specs=[_row_spec(288), _row_spec(288)],
        out_shape=[_sds((N, 288)), _sds((N, 288))],
    )(x, p["lin_l"]["W"], p["lin_l"]["b"].reshape(1, -1),
      p["lin_r"]["W"], p["lin_r"]["b"].reshape(1, -1))


def _tc_assemble_stats(gparts, bias):
    def body(g0, g1, g2, g3, b_ref, t_ref, s1_ref, s2_ref):
        t = _assemble8([g0[...], g1[...], g2[...], g3[...]]) + b_ref[...]
        t_ref[...] = t

        @pl.when(pl.program_id(0) == 0)
        def _():
            s1_ref[...] = jnp.zeros_like(s1_ref)
            s2_ref[...] = jnp.zeros_like(s2_ref)
        s1_ref[...] += jnp.sum(t, 0, keepdims=True)
        s2_ref[...] += jnp.sum(t * t, 0, keepdims=True)
    return pl.pallas_call(
        body, grid=(N // _BR,),
        in_specs=[_row_spec(128)] * 4 + [_one_spec(288)],
        out_specs=[_row_spec(288), _one_spec(288), _one_spec(288)],
        out_shape=[_sds((N, 288)), _sds((1, 288)), _sds((1, 288))],
    )(*gparts, bias.reshape(1, -1))


def _tc_gn_elu_stats(t, s1, s2, pn):
    def body(t_ref, s1_ref, s2_ref, ms, w, b, x_ref, o1_ref, o2_ref):
        x = _elu(_gn_from_stats(t_ref[...], s1_ref[...], s2_ref[...],
                                ms[...], w[...], b[...]))
        x_ref[...] = x

        @pl.when(pl.program_id(0) == 0)
        def _():
            o1_ref[...] = jnp.zeros_like(o1_ref)
            o2_ref[...] = jnp.zeros_like(o2_ref)
        o1_ref[...] += jnp.sum(x, 0, keepdims=True)
        o2_ref[...] += jnp.sum(x * x, 0, keepdims=True)
    return pl.pallas_call(
        body, grid=(N // _BR,),
        in_specs=[_row_spec(288)] + [_one_spec(288)] * 5,
        out_specs=[_row_spec(288), _one_spec(288), _one_spec(288)],
        out_shape=[_sds((N, 288)), _sds((1, 288)), _sds((1, 288))],
    )(t, s1, s2, pn["ms"].reshape(1, -1), pn["w"].reshape(1, -1),
      pn["b"].reshape(1, -1))


def _tc_gn_proj(x, s1, s2, pn, wcat, bcat):
    k = wcat.shape[1]

    def body(x_ref, s1_ref, s2_ref, ms, w, b, wc, bc, y_ref):
        xn = _gn_from_stats(x_ref[...], s1_ref[...], s2_ref[...],
                            ms[...], w[...], b[...])
        y_ref[...] = _dot(xn, wc[...]) + bc[...]
    return pl.pallas_call(
        body, grid=(N // _BR,),
        in_specs=[_row_spec(288)] + [_one_spec(288)] * 5
        + [pl.BlockSpec((288, k), lambda i: (0, 0)), _one_spec(k)],
        out_specs=[_row_spec(k)],
        out_shape=[_sds((N, k))],
    )(x, s1, s2, pn["ms"].reshape(1, -1), pn["w"].reshape(1, -1),
      pn["b"].reshape(1, -1), wcat, bcat.reshape(1, -1))[0]


def _tc_combine(gparts, tparts, skip, x_prev, bias_g, lp):
    def body(g0, g1, g2, g3, t0, t1, t2, t3, sk_ref, xp_ref, bg, lnw, lnb,
             wl, bl, xo_ref, s1_ref, s2_ref):
        xg = _elu(_assemble8([g0[...], g1[...], g2[...], g3[...]]) + bg[...])
        xt = _elu(_assemble8([t0[...], t1[...], t2[...], t3[...]])
                  + sk_ref[...])
        x = xg + xt + xp_ref[...]
        u = jnp.mean(x, -1, keepdims=True)
        d = x - u
        s = jnp.mean(d * d, -1, keepdims=True)
        ln = lnw[...] * d / jnp.sqrt(s + 1e-6) + lnb[...]
        y = _elu(_dot(ln, wl[...]) + bl[...])
        xo = y + x
        xo_ref[...] = xo

        @pl.when(pl.program_id(0) == 0)
        def _():
            s1_ref[...] = jnp.zeros_like(s1_ref)
            s2_ref[...] = jnp.zeros_like(s2_ref)
        s1_ref[...] += jnp.sum(xo, 0, keepdims=True)
        s2_ref[...] += jnp.sum(xo * xo, 0, keepdims=True)
    return pl.pallas_call(
        body, grid=(N // _BR,),
        in_specs=[_row_spec(128)] * 8 + [_row_spec(288)] * 2
        + [_one_spec(288)] * 3
        + [pl.BlockSpec((288, 288), lambda i: (0, 0)), _one_spec(288)],
        out_specs=[_row_spec(288), _one_spec(288), _one_spec(288)],
        out_shape=[_sds((N, 288)), _sds((1, 288)), _sds((1, 288))],
    )(*gparts, *tparts, skip, x_prev, bias_g.reshape(1, -1),
      lp["layer_norm"]["w"].reshape(1, -1), lp["layer_norm"]["b"].reshape(1, -1),
      lp["lin"]["W"], lp["lin"]["b"].reshape(1, -1))


def _tc_final(gg0, gg1, gt0, gt1, skip3, bias_g, lp, pfin_lin, pfin_norm,
              batch2d):
    def body(gg0_ref, gg1_ref, gt0_ref, gt1_ref, sk_ref, bg, lnw, lnb,
             wl, bl, wf, bf, fnw, fnb, b_ref, o_ref):
        gg = gg0_ref[...] + gg1_ref[...]
        gt = gt0_ref[...] + gt1_ref[...]
        xg = _elu(gg[:, 0:18] / (gg[:, 18:19] + 1e-16) + bg[...])
        xt = _elu(gt[:, 0:18] / (gt[:, 18:19] + 1e-16) + sk_ref[...])
        x = xg + xt
        res = x
        u = jnp.mean(x, -1, keepdims=True)
        d = x - u
        s = jnp.mean(d * d, -1, keepdims=True)
        ln = lnw[...] * d / jnp.sqrt(s + 1e-6) + lnb[...]
        y = _elu(_dot(ln, wl[...]) + bl[...])
        x = y + res
        onehot = (b_ref[...] == lax.broadcasted_iota(I32, (1, NG), 1)
                  ).astype(F32)
        pooled = jax.lax.dot_general(
            onehot, x, (((0,), (0,)), ((), ())),
            preferred_element_type=F32,
            precision=jax.lax.Precision.HIGHEST)
        out = _dot(pooled, wf[...]) + bf[...]
        u2 = jnp.mean(out, -1, keepdims=True)
        d2 = out - u2
        s2 = jnp.mean(d2 * d2, -1, keepdims=True)
        out = fnw[...] * d2 / jnp.sqrt(s2 + 1e-5) + fnb[...]
        o_ref[...] = _elu(out)
    return pl.pallas_call(
        body, out_shape=[_sds((NG, CH))],
    )(gg0, gg1, gt0, gt1, skip3, bias_g.reshape(1, -1),
      lp["layer_norm"]["w"].reshape(1, -1), lp["layer_norm"]["b"].reshape(1, -1),
      lp["lin"]["W"], lp["lin"]["b"].reshape(1, -1),
      pfin_lin["W"], pfin_lin["b"].reshape(1, -1),
      pfin_norm["w"].reshape(1, -1), pfin_norm["b"].reshape(1, -1),
      batch2d)[0]


def _parts_table(a288):
    t = a288.reshape(N, 4, 72).transpose(1, 0, 2)
    return jnp.pad(t, ((0, 0), (0, 0), (0, 56))).reshape(4 * N, 128)


def _pad128(a):
    return jnp.pad(a, ((0, 0), (0, 128 - a.shape[1])))


def _att_parts(att):
    t = att.reshape(4, 72)
    return jnp.pad(t, ((0, 0), (0, 8)))


def _we_parts(we):
    t = we.reshape(3, 4, 72).transpose(1, 0, 2)
    return jnp.pad(t, ((0, 0), (0, 0), (0, 8))).reshape(4, 240)


def _att_parts1(att):
    return jnp.pad(att.reshape(1, 18), ((0, 0), (0, 14)))


def _we_parts1(we):
    return jnp.pad(we.reshape(3, 18), ((0, 0), (0, 14))).reshape(1, 96)


def _out_parts(o):
    return [o[q * NPAD:q * NPAD + N] for q in range(4)]


def _cat_weights(lp):
    conv, tr = lp["conv"], lp["transformer"]
    ws = [conv["lin_l"]["W"], conv["lin_r"]["W"], tr["lin_q"]["W"],
          tr["lin_k"]["W"], tr["lin_v"]["W"], tr["lin_skip"]["W"]]
    bs = [conv["lin_l"]["b"], conv["lin_r"]["b"], tr["lin_q"]["b"],
          tr["lin_k"]["b"], tr["lin_v"]["b"], tr["lin_skip"]["b"]]
    return jnp.concatenate(ws, axis=1), jnp.concatenate(bs, axis=0)


def kernel(x, edge_index, edge_attr, batch, params):
    src = edge_index[0]
    dst = edge_index[1]

    def padded_edges(etot):
        pe = etot - E
        s_ = jnp.concatenate([src, jnp.zeros((pe,), I32)])
        d_ = jnp.concatenate([dst, jnp.zeros((pe,), I32)])
        a_ = jnp.pad(edge_attr, ((0, pe), (0, 13)))
        return s_, d_, a_

    eg8 = padded_edges(_mode_geom("gat8")[3])
    et8 = padded_edges(_mode_geom("trans8")[3])
    eg1 = padded_edges(_mode_geom("gat1")[3])
    et1 = padded_edges(_mode_geom("trans1")[3])
    zrows = jnp.zeros((RPT, 128), F32)

    def gat8(tabl, tabr, p):
        return _out_parts(_sc_conv(
            "gat8", _parts_table(tabl), _parts_table(tabr),
            *eg8, _att_parts(p["att"]),
            _we_parts(p["lin_edge"]["W"]), zrows))

    def trans8(tabq, tabk, tabv, p):
        return _out_parts(_sc_conv(
            "trans8", _parts_table(tabq), _parts_table(tabk),
            _parts_table(tabv), *et8,
            _we_parts(p["lin_edge"]["W"]), zrows))

    xl0, xr0 = _tc_initial_proj(x, params["initial_conv"])
    gp = gat8(xl0, xr0, params["initial_conv"])
    t, s1, s2 = _tc_assemble_stats(gp, params["initial_conv"]["bias"])
    xcur, s1, s2 = _tc_gn_elu_stats(t, s1, s2, params["initial_norm"])

    for i in range(NUM_LAYERS - 1):
        lp = params["layers"][i]
        wcat, bcat = _cat_weights(lp)
        y = _tc_gn_proj(xcur, s1, s2, lp["norm"], wcat, bcat)
        gp = gat8(y[:, 0:288], y[:, 288:576], lp["conv"])
        tp = trans8(y[:, 576:864], y[:, 864:1152], y[:, 1152:1440],
                    lp["transformer"])
        xcur, s1, s2 = _tc_combine(gp, tp, y[:, 1440:1728], xcur,
                                   lp["conv"]["bias"], lp)

    lp = params["layers"][NUM_LAYERS - 1]
    wcat, bcat = _cat_weights(lp)
    y = _tc_gn_proj(xcur, s1, s2, lp["norm"], wcat, bcat)

    og = _sc_conv("gat1", _pad128(y[:, 0:18]), _pad128(y[:, 18:36]),
                  *eg1, _att_parts1(lp["conv"]["att"]),
                  _we_parts1(lp["conv"]["lin_edge"]["W"]), zrows)
    ot = _sc_conv("trans1", _pad128(y[:, 36:54]), _pad128(y[:, 54:72]),
                  _pad128(y[:, 72:90]), *et1,
                  _we_parts1(lp["transformer"]["lin_edge"]["W"]), zrows)

    return _tc_final(og[0:N], og[NPAD:NPAD + N], ot[0:N], ot[NPAD:NPAD + N],
                     y[:, 90:108], lp["conv"]["bias"], lp,
                     params["final_lin"], params["final_norm"],
                     batch.reshape(-1, 1))
